# Initial kernel scaffold; baseline (speedup 1.0000x reference)
import jax
import jax.numpy as jnp
from jax.experimental import pallas as pl


def kernel(x, w_cat, s_cat, b_cat, m_cat, w2, s2, b2, m2, wfc1, sf1, bf1, wfc2, bfc2):
    raise NotImplementedError("write your pallas kernel here")


# speedup vs baseline: 9.7679x; 9.7679x over previous
import functools

import jax
import jax.numpy as jnp
from jax import lax
from jax.experimental import pallas as pl
from jax.experimental.pallas import tpu as pltpu


def _conv3x3(xf, wk, W):
    HW, Cin = xf.shape
    dt = xf.dtype
    j = lax.broadcasted_iota(jnp.int32, (HW, 1), 0) % W
    z1 = jnp.zeros((1, Cin), dt)
    shl = jnp.where(j != (W - 1), jnp.concatenate([xf[1:], z1], axis=0), jnp.zeros((), dt))
    shr = jnp.where(j != 0, jnp.concatenate([z1, xf[:-1]], axis=0), jnp.zeros((), dt))
    cc = jnp.concatenate([shr, xf, shl], axis=-1)
    zW = jnp.zeros((W, 3 * Cin), dt)
    up = jnp.concatenate([cc[W:], zW], axis=0)
    down = jnp.concatenate([zW, cc[:-W]], axis=0)
    acc = jnp.dot(down, wk[0], preferred_element_type=jnp.float32)
    acc += jnp.dot(cc, wk[1], preferred_element_type=jnp.float32)
    acc += jnp.dot(up, wk[2], preferred_element_type=jnp.float32)
    return acc


def _block_kernel(x_ref, wk1_ref, s1_ref, b1_ref, mc_ref, wk2_ref, s2_ref,
                  b2_ref, m2_ref, wfc1_ref, sf1_ref, bf1_ref, wfc2_ref,
                  bfc2_ref, o_ref, *, W, C):
    xf = x_ref[0]

    acc1 = _conv3x3(xf, wk1_ref[...], W)
    y = acc1 * s1_ref[...] + b1_ref[...]
    y = jnp.where(mc_ref[...] > 0.0, jnp.maximum(y, 0.0), y)
    y1 = y[:, :C].astype(xf.dtype)
    res = y[:, C:]

    acc2 = _conv3x3(y1, wk2_ref[...], W)
    x2 = acc2 * s2_ref[...] + b2_ref[...]
    x2 = jnp.where(m2_ref[...] > 0.0, jnp.maximum(x2, 0.0), x2)

    HW = xf.shape[0]
    gap = jnp.sum(x2[:, :C] + x2[:, C:], axis=0, keepdims=True) * (1.0 / HW)
    g1 = jnp.dot(gap, wfc1_ref[...], preferred_element_type=jnp.float32)
    g1 = jnp.maximum(g1 * sf1_ref[...] + bf1_ref[...], 0.0)
    a = jnp.dot(g1, wfc2_ref[...], preferred_element_type=jnp.float32)
    a = a + bfc2_ref[...]

    Cq = C // 2
    a00, a01 = a[:, 0:Cq], a[:, Cq:2 * Cq]
    a10, a11 = a[:, 2 * Cq:3 * Cq], a[:, 3 * Cq:4 * Cq]
    m0 = jnp.maximum(a00, a01)
    e00, e01 = jnp.exp(a00 - m0), jnp.exp(a01 - m0)
    r0 = 1.0 / (e00 + e01)
    m1 = jnp.maximum(a10, a11)
    e10, e11 = jnp.exp(a10 - m1), jnp.exp(a11 - m1)
    r1 = 1.0 / (e10 + e11)
    attn = jnp.concatenate([e00 * r0, e10 * r1, e01 * r0, e11 * r1], axis=-1)

    w = x2 * attn
    s = jnp.maximum(w[:, :C] + w[:, C:], 0.0)
    o_ref[0] = jnp.maximum(s + res, 0.0).astype(o_ref.dtype)


def kernel(x, w_cat, s_cat, b_cat, m_cat, w2, s2, b2, m2, wfc1, sf1, bf1,
           wfc2, bfc2):
    B, Cin, H, W = x.shape
    C = w_cat.shape[1] // 2
    HW = H * W

    x3 = jnp.transpose(x, (0, 2, 3, 1)).reshape(B, HW, Cin).astype(jnp.bfloat16)
    wk1 = w_cat.reshape(3, 3 * Cin, 2 * C).astype(jnp.bfloat16)
    wk2 = w2.reshape(3, 3 * C, 2 * C).astype(jnp.bfloat16)

    kern = functools.partial(_block_kernel, W=W, C=C)
    const = lambda *_: (0, 0)
    const3 = lambda *_: (0, 0, 0)
    out = pl.pallas_call(
        kern,
        out_shape=jax.ShapeDtypeStruct((B, HW, C), jnp.float32),
        grid=(B,),
        in_specs=[
            pl.BlockSpec((1, HW, Cin), lambda b: (b, 0, 0)),
            pl.BlockSpec((3, 3 * Cin, 2 * C), const3),
            pl.BlockSpec((1, 2 * C), const),
            pl.BlockSpec((1, 2 * C), const),
            pl.BlockSpec((1, 2 * C), const),
            pl.BlockSpec((3, 3 * C, 2 * C), const3),
            pl.BlockSpec((1, 2 * C), const),
            pl.BlockSpec((1, 2 * C), const),
            pl.BlockSpec((1, 2 * C), const),
            pl.BlockSpec(wfc1.shape, const),
            pl.BlockSpec(sf1.shape, const),
            pl.BlockSpec(bf1.shape, const),
            pl.BlockSpec(wfc2.shape, const),
            pl.BlockSpec(bfc2.shape, const),
        ],
        out_specs=pl.BlockSpec((1, HW, C), lambda b: (b, 0, 0)),
        compiler_params=pltpu.CompilerParams(dimension_semantics=("parallel",)),
    )(x3, wk1, s1_2d(s_cat), s1_2d(b_cat), s1_2d(m_cat), wk2, s1_2d(s2),
      s1_2d(b2), s1_2d(m2), wfc1, s1_2d(sf1), s1_2d(bf1), wfc2, s1_2d(bfc2))

    return jnp.transpose(out.reshape(B, H, W, C), (0, 3, 1, 2))


def s1_2d(v):
    return v.reshape(1, -1)


# speedup vs baseline: 11.2329x; 1.1500x over previous
import functools

import jax
import jax.numpy as jnp
from jax import lax
from jax.experimental import pallas as pl
from jax.experimental.pallas import tpu as pltpu


def _conv3x3(xf, wk, NI, H, W):
    M, Cin = xf.shape
    HW = H * W
    dt = xf.dtype
    j = lax.broadcasted_iota(jnp.int32, (M, 1), 0) % W
    z1 = jnp.zeros((1, Cin), dt)
    shl = jnp.where(j != (W - 1), jnp.concatenate([xf[1:], z1], axis=0), jnp.zeros((), dt))
    shr = jnp.where(j != 0, jnp.concatenate([z1, xf[:-1]], axis=0), jnp.zeros((), dt))
    cc = jnp.concatenate([shr, xf, shl], axis=-1)
    cc3 = cc.reshape(NI, HW, 3 * Cin)
    zW = jnp.zeros((NI, W, 3 * Cin), dt)
    up = jnp.concatenate([cc3[:, W:], zW], axis=1).reshape(M, 3 * Cin)
    down = jnp.concatenate([zW, cc3[:, :HW - W]], axis=1).reshape(M, 3 * Cin)
    acc = jnp.dot(down, wk[0], preferred_element_type=jnp.float32)
    acc += jnp.dot(cc, wk[1], preferred_element_type=jnp.float32)
    acc += jnp.dot(up, wk[2], preferred_element_type=jnp.float32)
    return acc


def _block_kernel(x_ref, wk1_ref, s1_ref, b1_ref, mc_ref, wk2_ref, s2_ref,
                  b2_ref, m2_ref, wfc1_ref, sf1_ref, bf1_ref, wfc2_ref,
                  bfc2_ref, o_ref, *, NI, H, W, C):
    HW = H * W
    M = NI * HW
    xf = x_ref[...].reshape(M, x_ref.shape[-1])

    acc1 = _conv3x3(xf, wk1_ref[...], NI, H, W)
    y = acc1 * s1_ref[...] + b1_ref[...]
    y = jnp.where(mc_ref[...] > 0.0, jnp.maximum(y, 0.0), y)
    y1 = y[:, :C].astype(xf.dtype)
    res = y[:, C:]

    acc2 = _conv3x3(y1, wk2_ref[...], NI, H, W)
    x2 = acc2 * s2_ref[...] + b2_ref[...]
    x2 = jnp.where(m2_ref[...] > 0.0, jnp.maximum(x2, 0.0), x2)

    rsum = x2[:, :C] + x2[:, C:]
    gap = jnp.sum(rsum.reshape(NI, HW, C), axis=1) * (1.0 / HW)
    g1 = jnp.dot(gap, wfc1_ref[...], preferred_element_type=jnp.float32)
    g1 = jnp.maximum(g1 * sf1_ref[...] + bf1_ref[...], 0.0)
    a = jnp.dot(g1, wfc2_ref[...], preferred_element_type=jnp.float32)
    a = a + bfc2_ref[...]

    Cq = C // 2
    a00, a01 = a[:, 0:Cq], a[:, Cq:2 * Cq]
    a10, a11 = a[:, 2 * Cq:3 * Cq], a[:, 3 * Cq:4 * Cq]
    m0 = jnp.maximum(a00, a01)
    e00, e01 = jnp.exp(a00 - m0), jnp.exp(a01 - m0)
    r0 = 1.0 / (e00 + e01)
    m1 = jnp.maximum(a10, a11)
    e10, e11 = jnp.exp(a10 - m1), jnp.exp(a11 - m1)
    r1 = 1.0 / (e10 + e11)
    attn = jnp.concatenate([e00 * r0, e10 * r1, e01 * r0, e11 * r1], axis=-1)

    w = x2.reshape(NI, HW, 2 * C) * attn.reshape(NI, 1, 2 * C)
    w = w.reshape(M, 2 * C)
    s = jnp.maximum(w[:, :C] + w[:, C:], 0.0)
    out = jnp.maximum(s + res, 0.0)
    o_ref[...] = out.reshape(NI, HW, C).astype(o_ref.dtype)


def _s1_2d(v):
    return v.reshape(1, -1)


def kernel(x, w_cat, s_cat, b_cat, m_cat, w2, s2, b2, m2, wfc1, sf1, bf1,
           wfc2, bfc2):
    B, Cin, H, W = x.shape
    C = w_cat.shape[1] // 2
    HW = H * W
    NI = 4 if B % 4 == 0 else 1

    x3 = jnp.transpose(x, (0, 2, 3, 1)).reshape(B, HW, Cin).astype(jnp.bfloat16)
    wk1 = w_cat.reshape(3, 3 * Cin, 2 * C).astype(jnp.bfloat16)
    wk2 = w2.reshape(3, 3 * C, 2 * C).astype(jnp.bfloat16)

    kern = functools.partial(_block_kernel, NI=NI, H=H, W=W, C=C)
    const = lambda *_: (0, 0)
    const3 = lambda *_: (0, 0, 0)
    out = pl.pallas_call(
        kern,
        out_shape=jax.ShapeDtypeStruct((B, HW, C), jnp.float32),
        grid=(B // NI,),
        in_specs=[
            pl.BlockSpec((NI, HW, Cin), lambda b: (b, 0, 0)),
            pl.BlockSpec((3, 3 * Cin, 2 * C), const3),
            pl.BlockSpec((1, 2 * C), const),
            pl.BlockSpec((1, 2 * C), const),
            pl.BlockSpec((1, 2 * C), const),
            pl.BlockSpec((3, 3 * C, 2 * C), const3),
            pl.BlockSpec((1, 2 * C), const),
            pl.BlockSpec((1, 2 * C), const),
            pl.BlockSpec((1, 2 * C), const),
            pl.BlockSpec(wfc1.shape, const),
            pl.BlockSpec(sf1.shape, const),
            pl.BlockSpec(bf1.shape, const),
            pl.BlockSpec(wfc2.shape, const),
            pl.BlockSpec(bfc2.shape, const),
        ],
        out_specs=pl.BlockSpec((NI, HW, C), lambda b: (b, 0, 0)),
        compiler_params=pltpu.CompilerParams(dimension_semantics=("parallel",)),
    )(x3, wk1, _s1_2d(s_cat), _s1_2d(b_cat), _s1_2d(m_cat), wk2, _s1_2d(s2),
      _s1_2d(b2), _s1_2d(m2), wfc1, _s1_2d(sf1), _s1_2d(bf1), wfc2,
      _s1_2d(bfc2))

    return jnp.transpose(out.reshape(B, H, W, C), (0, 3, 1, 2))


# speedup vs baseline: 11.7729x; 1.0481x over previous
import functools

import jax
import jax.numpy as jnp
from jax import lax
from jax.experimental import pallas as pl
from jax.experimental.pallas import tpu as pltpu


def _conv3x3_chw(xT, wT, W):
    Cin, HW = xT.shape
    dt = xT.dtype
    jl = lax.broadcasted_iota(jnp.int32, (1, HW), 1) % W
    blocks = []
    for kh in range(3):
        for kw in range(3):
            s = (kh - 1) * W + (kw - 1)
            if s > 0:
                sh = jnp.concatenate([xT[:, s:], jnp.zeros((Cin, s), dt)], axis=1)
            elif s < 0:
                sh = jnp.concatenate([jnp.zeros((Cin, -s), dt), xT[:, :HW + s]], axis=1)
            else:
                sh = xT
            if kw == 0:
                sh = jnp.where(jl != 0, sh, jnp.zeros((), dt))
            elif kw == 2:
                sh = jnp.where(jl != W - 1, sh, jnp.zeros((), dt))
            blocks.append(sh)
    ccT = jnp.concatenate(blocks, axis=0)
    return jnp.dot(wT, ccT, preferred_element_type=jnp.float32)


def _block_kernel(x_ref, w1_ref, s1_ref, b1_ref, mc_ref, w2_ref, s2_ref,
                  b2_ref, m2_ref, wfc1_ref, sf1_ref, bf1_ref, wfc2_ref,
                  bfc2_ref, o_ref, *, NI, W, C):
    HW = x_ref.shape[-1]
    Cq = C // 2
    for i in range(NI):
        xT = x_ref[i].astype(jnp.bfloat16)

        acc1 = _conv3x3_chw(xT, w1_ref[...], W)
        y = acc1 * s1_ref[...] + b1_ref[...]
        y = jnp.where(mc_ref[...] > 0.0, jnp.maximum(y, 0.0), y)
        y1 = y[:C].astype(jnp.bfloat16)
        res = y[C:]

        acc2 = _conv3x3_chw(y1, w2_ref[...], W)
        x2 = acc2 * s2_ref[...] + b2_ref[...]
        x2 = jnp.where(m2_ref[...] > 0.0, jnp.maximum(x2, 0.0), x2)

        rsum = x2[:C] + x2[C:]
        ones = jnp.ones((HW, 1), jnp.float32)
        gap = jnp.dot(rsum, ones, preferred_element_type=jnp.float32) * (1.0 / HW)
        g1 = jnp.dot(wfc1_ref[...], gap, preferred_element_type=jnp.float32)
        g1 = jnp.maximum(g1 * sf1_ref[...] + bf1_ref[...], 0.0)
        a = jnp.dot(wfc2_ref[...], g1, preferred_element_type=jnp.float32)
        a = a + bfc2_ref[...]

        a00, a01 = a[0:Cq], a[Cq:2 * Cq]
        a10, a11 = a[2 * Cq:3 * Cq], a[3 * Cq:4 * Cq]
        m0 = jnp.maximum(a00, a01)
        e00, e01 = jnp.exp(a00 - m0), jnp.exp(a01 - m0)
        r0 = 1.0 / (e00 + e01)
        m1 = jnp.maximum(a10, a11)
        e10, e11 = jnp.exp(a10 - m1), jnp.exp(a11 - m1)
        r1 = 1.0 / (e10 + e11)
        attn = jnp.concatenate([e00 * r0, e10 * r1, e01 * r0, e11 * r1], axis=0)

        w = x2 * attn
        s = jnp.maximum(w[:C] + w[C:], 0.0)
        o_ref[i] = jnp.maximum(s + res, 0.0).astype(o_ref.dtype)


def _col(v):
    return v.reshape(-1, 1)


def kernel(x, w_cat, s_cat, b_cat, m_cat, w2, s2, b2, m2, wfc1, sf1, bf1,
           wfc2, bfc2):
    B, Cin, H, W = x.shape
    C = w_cat.shape[1] // 2
    HW = H * W
    NI = 4 if B % 4 == 0 else 1

    xc = x.reshape(B, Cin, HW)
    w1T = w_cat.T.astype(jnp.bfloat16)
    w2T = w2.T.astype(jnp.bfloat16)

    kern = functools.partial(_block_kernel, NI=NI, W=W, C=C)
    const = lambda *_: (0, 0)
    out = pl.pallas_call(
        kern,
        out_shape=jax.ShapeDtypeStruct((B, C, HW), jnp.float32),
        grid=(B // NI,),
        in_specs=[
            pl.BlockSpec((NI, Cin, HW), lambda b: (b, 0, 0)),
            pl.BlockSpec(w1T.shape, const),
            pl.BlockSpec((2 * C, 1), const),
            pl.BlockSpec((2 * C, 1), const),
            pl.BlockSpec((2 * C, 1), const),
            pl.BlockSpec(w2T.shape, const),
            pl.BlockSpec((2 * C, 1), const),
            pl.BlockSpec((2 * C, 1), const),
            pl.BlockSpec((2 * C, 1), const),
            pl.BlockSpec(wfc1.T.shape, const),
            pl.BlockSpec((wfc1.shape[1], 1), const),
            pl.BlockSpec((wfc1.shape[1], 1), const),
            pl.BlockSpec(wfc2.T.shape, const),
            pl.BlockSpec((wfc2.shape[1], 1), const),
        ],
        out_specs=pl.BlockSpec((NI, C, HW), lambda b: (b, 0, 0)),
        compiler_params=pltpu.CompilerParams(dimension_semantics=("parallel",)),
    )(xc, w1T, _col(s_cat), _col(b_cat), _col(m_cat), w2T, _col(s2),
      _col(b2), _col(m2), wfc1.T, _col(sf1), _col(bf1), wfc2.T, _col(bfc2))

    return out.reshape(B, C, H, W)
